# Initial kernel scaffold; baseline (speedup 1.0000x reference)
import jax
import jax.numpy as jnp
from jax.experimental import pallas as pl


def kernel(x, adjc, adjc_mask, coordinates):
    raise NotImplementedError("write your pallas kernel here")


# speedup vs baseline: 3.4109x; 3.4109x over previous
import functools

import jax
import jax.numpy as jnp
from jax import lax
from jax.experimental import pallas as pl
from jax.experimental.pallas import tpu as pltpu
from jax.experimental.pallas import tpu_sc as plsc

N_NODES = 65536
NH = 9
D_FEAT = 128

NC = 2
NS = 16
NW = NC * NS

TOTAL = N_NODES * NH
B_PER_W = TOTAL // NW
CHUNK = 128
NCHUNK = B_PER_W // CHUNK
NBUF = 4


def _gather_kernel(table_hbm, idx_hbm, out_hbm, idx_v, bufs, gsems, ssems):
    wid = lax.axis_index("s") * NC + lax.axis_index("c")

    pltpu.sync_copy(idx_hbm.at[wid], idx_v)

    def start_gather(j, b):
        return pltpu.async_copy(table_hbm.at[idx_v.at[j]], bufs.at[b], gsems.at[b])

    def start_scatter(j, b):
        return pltpu.async_copy(bufs.at[b], out_hbm.at[wid, j], ssems.at[b])

    for b in range(NBUF):
        start_gather(b, b)

    def group(g, carry):
        j0 = g * NBUF
        for b in range(NBUF):
            j = j0 + b
            pltpu.make_async_copy(table_hbm.at[idx_v.at[j]], bufs.at[b],
                                  gsems.at[b]).wait()
            start_scatter(j, b)
            nj = j + NBUF

            @pl.when(nj < NCHUNK)
            def _():
                pltpu.make_async_copy(bufs.at[b], out_hbm.at[wid, j],
                                      ssems.at[b]).wait()
                start_gather(nj, b)

        return carry

    lax.fori_loop(0, NCHUNK // NBUF, group, 0)

    for b in range(NBUF):
        j = NCHUNK - NBUF + b
        pltpu.make_async_copy(bufs.at[b], out_hbm.at[wid, j],
                              ssems.at[b]).wait()


@jax.jit
def _run(table, idx):
    mesh = plsc.VectorSubcoreMesh(core_axis_name="c", subcore_axis_name="s",
                                  num_cores=NC, num_subcores=NS)
    kern = pl.kernel(
        _gather_kernel,
        out_type=jax.ShapeDtypeStruct((NW, NCHUNK, CHUNK, D_FEAT), jnp.float32),
        mesh=mesh,
        scratch_types=[
            pltpu.VMEM((NCHUNK, CHUNK), jnp.int32),
            pltpu.VMEM((NBUF, CHUNK, D_FEAT), jnp.float32),
            pltpu.SemaphoreType.DMA((NBUF,)),
            pltpu.SemaphoreType.DMA((NBUF,)),
        ],
    )
    return kern(table, idx)


def kernel(x, adjc, adjc_mask, coordinates):
    table = x.reshape(N_NODES, D_FEAT)
    idx = adjc.reshape(NW, NCHUNK, CHUNK)
    out = _run(table, idx)
    return out.reshape(1, 1, N_NODES, NH, D_FEAT)
